# baseline (speedup 1.0000x reference)
import functools

import jax
import jax.numpy as jnp
from jax import lax
from jax.experimental import pallas as pl
from jax.experimental.pallas import tpu as pltpu
from jax.experimental.pallas import tpu_sc as plsc

DIM = 64
NCORES = 2
NSUB = 16
LANES = 16
NW = NCORES * NSUB

BM = 25000
CHUNK = 192
GROUPS = CHUNK // LANES


def _project_kernel(z_ref, w_ref, out_ref):
    out_ref[...] = jnp.dot(z_ref[...], w_ref[...],
                           preferred_element_type=jnp.float32)


def _build_table(z, W):
    m, k = z.shape
    bm = BM if m % BM == 0 else m
    zw = pl.pallas_call(
        _project_kernel,
        grid=(m // bm,),
        in_specs=[pl.BlockSpec((bm, k), lambda i: (i, 0)),
                  pl.BlockSpec((k, k), lambda i: (0, 0))],
        out_specs=pl.BlockSpec((bm, k), lambda i: (i, 0)),
        out_shape=jax.ShapeDtypeStruct((m, k), jnp.float32),
    )(z, W)
    return jnp.concatenate([zw, z], axis=1)


@functools.cache
def _make_edge_kernel(E):
    per_w = E // NW
    assert per_w * NW == E and per_w % 8 == 0 and per_w >= CHUNK
    n_chunks = -(-per_w // CHUNK)
    assert (per_w - CHUNK) % 8 == 0
    n_pairs = (n_chunks + 1) // 2
    last = 2 * n_pairs - 1

    mesh = plsc.VectorSubcoreMesh(core_axis_name="c", subcore_axis_name="s")

    def body(tab_hbm, edge_hbm, bias_hbm, out_hbm,
             sidx0, sidx1, didx0, didx1, srow0, srow1, drow0, drow1,
             out_v, bias_v,
             sem_i0, sem_i1, sem_g0, sem_g1):
        sidx = (sidx0, sidx1)
        didx = (didx0, didx1)
        srow = (srow0, srow1)
        drow = (drow0, drow1)
        sem_i = (sem_i0, sem_i1)
        sem_g = (sem_g0, sem_g1)

        wid = lax.axis_index("s") * NCORES + lax.axis_index("c")
        base = wid * per_w
        pltpu.sync_copy(bias_hbm, bias_v)
        bias_vec = bias_v[...]
        lane = lax.iota(jnp.int32, LANES)

        def start_of(c):
            return jnp.minimum(c * CHUNK, per_w - CHUNK)

        def issue_idx(c, b):
            s = base + start_of(c)
            pltpu.async_copy(edge_hbm.at[pl.ds(s, CHUNK)], sidx[b], sem_i[b])
            pltpu.async_copy(edge_hbm.at[pl.ds(E + s, CHUNK)], didx[b],
                             sem_i[b])

        def wait_idx(b):
            pltpu.make_async_copy(edge_hbm.at[pl.ds(0, CHUNK)],
                                  sidx[b], sem_i[b]).wait()
            pltpu.make_async_copy(edge_hbm.at[pl.ds(0, CHUNK)],
                                  didx[b], sem_i[b]).wait()

        def issue_gather(b):
            pltpu.async_copy(tab_hbm.at[sidx[b]], srow[b], sem_g[b])
            pltpu.async_copy(tab_hbm.at[didx[b]], drow[b], sem_g[b])

        def wait_gather(b):
            pltpu.make_async_copy(tab_hbm.at[pl.ds(0, CHUNK)],
                                  srow[b], sem_g[b]).wait()
            pltpu.make_async_copy(tab_hbm.at[pl.ds(0, CHUNK)],
                                  drow[b], sem_g[b]).wait()

        masks = {k: (lane & k) == 0 for k in (1, 2, 4, 8)}

        def compute(c, b):
            s_loc = start_of(c)

            def group(g, carry):
                ps = []
                for e in range(LANES):
                    row = g * LANES + e
                    s = (srow[b][row, pl.ds(0, LANES)]
                         * drow[b][row, pl.ds(DIM, LANES)])
                    for q in range(1, 4):
                        s = s + (srow[b][row, pl.ds(q * LANES, LANES)]
                                 * drow[b][row, pl.ds(DIM + q * LANES, LANES)])
                    ps.append(s)
                k = 1
                while len(ps) > 1:
                    mk = masks[k]
                    perm = lane ^ k
                    ps = [jnp.where(mk, ps[j], ps[j + 1])
                          + jnp.take(jnp.where(mk, ps[j + 1], ps[j]), perm)
                          for j in range(0, len(ps), 2)]
                    k *= 2
                out_v[pl.ds(s_loc + g * LANES, LANES)] = ps[0] + bias_vec
                return carry

            lax.fori_loop(0, GROUPS, group, 0)

        issue_idx(0, 0)
        wait_idx(0)
        issue_gather(0)
        issue_idx(1, 1)

        def pair(p, carry):
            for b in range(2):
                c = 2 * p + b
                wait_idx(1 - b)
                issue_gather(1 - b)
                wait_gather(b)
                issue_idx(jnp.minimum(c + 2, last), b)
                compute(c, b)
            return carry

        lax.fori_loop(0, n_pairs, pair, 0)

        wait_gather(0)
        wait_idx(1)

        pltpu.sync_copy(out_v, out_hbm.at[pl.ds(base, per_w)])

    return pl.kernel(
        body,
        out_type=jax.ShapeDtypeStruct((E,), jnp.float32),
        mesh=mesh,
        scratch_types=[
            pltpu.VMEM((CHUNK,), jnp.int32),
            pltpu.VMEM((CHUNK,), jnp.int32),
            pltpu.VMEM((CHUNK,), jnp.int32),
            pltpu.VMEM((CHUNK,), jnp.int32),
            pltpu.VMEM((CHUNK, 2 * DIM), jnp.float32),
            pltpu.VMEM((CHUNK, 2 * DIM), jnp.float32),
            pltpu.VMEM((CHUNK, 2 * DIM), jnp.float32),
            pltpu.VMEM((CHUNK, 2 * DIM), jnp.float32),
            pltpu.VMEM((E // NW,), jnp.float32),
            pltpu.VMEM((LANES,), jnp.float32),
            pltpu.SemaphoreType.DMA,
            pltpu.SemaphoreType.DMA,
            pltpu.SemaphoreType.DMA,
            pltpu.SemaphoreType.DMA,
        ],
    )


def kernel(z, edge_index, W, bias):
    table = _build_table(z, W)
    if edge_index.dtype != jnp.int32:
        edge_index = edge_index.astype(jnp.int32)
    bias16 = jnp.broadcast_to(bias.astype(jnp.float32), (LANES,))
    edge_fn = _make_edge_kernel(edge_index.shape[1])
    return edge_fn(table, edge_index.reshape(-1), bias16)


# speedup vs baseline: 1.0656x; 1.0656x over previous
import functools

import jax
import jax.numpy as jnp
from jax import lax
from jax.experimental import pallas as pl
from jax.experimental.pallas import tpu as pltpu
from jax.experimental.pallas import tpu_sc as plsc

DIM = 64
NCORES = 2
NSUB = 16
LANES = 16
NW = NCORES * NSUB

BM = 25000
CHUNK = 192
GROUPS = CHUNK // LANES


def _table_kernel(z_ref, w_ref, out_ref):
    out_ref[:, :DIM] = jnp.dot(z_ref[...], w_ref[...],
                               preferred_element_type=jnp.float32)
    out_ref[:, DIM:] = z_ref[...]


def _build_table(z, W):
    m, k = z.shape
    bm = BM if m % BM == 0 else m
    return pl.pallas_call(
        _table_kernel,
        grid=(m // bm,),
        in_specs=[pl.BlockSpec((bm, k), lambda i: (i, 0)),
                  pl.BlockSpec((k, k), lambda i: (0, 0))],
        out_specs=pl.BlockSpec((bm, 2 * k), lambda i: (i, 0)),
        out_shape=jax.ShapeDtypeStruct((m, 2 * k), jnp.float32),
    )(z, W)


@functools.cache
def _make_edge_kernel(E):
    per_w = E // NW
    assert per_w * NW == E and per_w % 8 == 0 and per_w >= CHUNK
    n_chunks = -(-per_w // CHUNK)
    assert (per_w - CHUNK) % 8 == 0
    n_pairs = (n_chunks + 1) // 2
    last = 2 * n_pairs - 1

    mesh = plsc.VectorSubcoreMesh(core_axis_name="c", subcore_axis_name="s")

    def body(tab_hbm, edge_hbm, bias_hbm, out_hbm,
             sidx0, sidx1, didx0, didx1, srow0, srow1, drow0, drow1,
             out_v, bias_v,
             sem_i0, sem_i1, sem_g0, sem_g1):
        sidx = (sidx0, sidx1)
        didx = (didx0, didx1)
        srow = (srow0, srow1)
        drow = (drow0, drow1)
        sem_i = (sem_i0, sem_i1)
        sem_g = (sem_g0, sem_g1)

        wid = lax.axis_index("s") * NCORES + lax.axis_index("c")
        base = wid * per_w
        pltpu.sync_copy(bias_hbm, bias_v)
        bias_vec = bias_v[...]
        lane = lax.iota(jnp.int32, LANES)

        def start_of(c):
            return jnp.minimum(c * CHUNK, per_w - CHUNK)

        def issue_idx(c, b):
            s = base + start_of(c)
            pltpu.async_copy(edge_hbm.at[pl.ds(s, CHUNK)], sidx[b], sem_i[b])
            pltpu.async_copy(edge_hbm.at[pl.ds(E + s, CHUNK)], didx[b],
                             sem_i[b])

        def wait_idx(b):
            pltpu.make_async_copy(edge_hbm.at[pl.ds(0, CHUNK)],
                                  sidx[b], sem_i[b]).wait()
            pltpu.make_async_copy(edge_hbm.at[pl.ds(0, CHUNK)],
                                  didx[b], sem_i[b]).wait()

        def issue_gather(b):
            pltpu.async_copy(tab_hbm.at[sidx[b]], srow[b], sem_g[b])
            pltpu.async_copy(tab_hbm.at[didx[b]], drow[b], sem_g[b])

        def wait_gather(b):
            pltpu.make_async_copy(tab_hbm.at[pl.ds(0, CHUNK)],
                                  srow[b], sem_g[b]).wait()
            pltpu.make_async_copy(tab_hbm.at[pl.ds(0, CHUNK)],
                                  drow[b], sem_g[b]).wait()

        masks = {k: (lane & k) == 0 for k in (1, 2, 4, 8)}

        def compute(c, b):
            s_loc = start_of(c)

            def group(g, carry):
                ps = []
                for e in range(LANES):
                    row = g * LANES + e
                    s = (srow[b][row, pl.ds(0, LANES)]
                         * drow[b][row, pl.ds(DIM, LANES)])
                    for q in range(1, 4):
                        s = s + (srow[b][row, pl.ds(q * LANES, LANES)]
                                 * drow[b][row, pl.ds(DIM + q * LANES, LANES)])
                    ps.append(s)
                k = 1
                while len(ps) > 1:
                    mk = masks[k]
                    perm = lane ^ k
                    ps = [jnp.where(mk, ps[j], ps[j + 1])
                          + jnp.take(jnp.where(mk, ps[j + 1], ps[j]), perm)
                          for j in range(0, len(ps), 2)]
                    k *= 2
                out_v[pl.ds(s_loc + g * LANES, LANES)] = ps[0] + bias_vec
                return carry

            lax.fori_loop(0, GROUPS, group, 0)

        issue_idx(0, 0)
        wait_idx(0)
        issue_gather(0)
        issue_idx(1, 1)

        def pair(p, carry):
            for b in range(2):
                c = 2 * p + b
                wait_idx(1 - b)
                issue_gather(1 - b)
                wait_gather(b)
                issue_idx(jnp.minimum(c + 2, last), b)
                compute(c, b)
            return carry

        lax.fori_loop(0, n_pairs, pair, 0)

        wait_gather(0)
        wait_idx(1)

        pltpu.sync_copy(out_v, out_hbm.at[pl.ds(base, per_w)])

    return pl.kernel(
        body,
        out_type=jax.ShapeDtypeStruct((E,), jnp.float32),
        mesh=mesh,
        scratch_types=[
            pltpu.VMEM((CHUNK,), jnp.int32),
            pltpu.VMEM((CHUNK,), jnp.int32),
            pltpu.VMEM((CHUNK,), jnp.int32),
            pltpu.VMEM((CHUNK,), jnp.int32),
            pltpu.VMEM((CHUNK, 2 * DIM), jnp.float32),
            pltpu.VMEM((CHUNK, 2 * DIM), jnp.float32),
            pltpu.VMEM((CHUNK, 2 * DIM), jnp.float32),
            pltpu.VMEM((CHUNK, 2 * DIM), jnp.float32),
            pltpu.VMEM((E // NW,), jnp.float32),
            pltpu.VMEM((LANES,), jnp.float32),
            pltpu.SemaphoreType.DMA,
            pltpu.SemaphoreType.DMA,
            pltpu.SemaphoreType.DMA,
            pltpu.SemaphoreType.DMA,
        ],
    )


def kernel(z, edge_index, W, bias):
    table = _build_table(z, W)
    if edge_index.dtype != jnp.int32:
        edge_index = edge_index.astype(jnp.int32)
    bias16 = jnp.broadcast_to(bias.astype(jnp.float32), (LANES,))
    edge_fn = _make_edge_kernel(edge_index.shape[1])
    return edge_fn(table, edge_index.reshape(-1), bias16)
